# Initial kernel scaffold; baseline (speedup 1.0000x reference)
import jax
import jax.numpy as jnp
from jax.experimental import pallas as pl


def kernel(x_joint, x_torso, Wj, bj, Wt, bt, W1_tj_rel, b1_tj, W1_tj_root, W1_jj_rel, b1_jj, W1_jj_root, W1_jt_rel, b1_jt, W1_jt_root, W2_tj_rel, b2_tj, W2_tj_root, W2_jj_rel, b2_jj, W2_jj_root, W2_jt_rel, b2_jt, W2_jt_root, Wout, bout, ei_tj, ei_jt, ei_jj):
    raise NotImplementedError("write your pallas kernel here")


# speedup vs baseline: 11.3364x; 11.3364x over previous
import jax
import jax.numpy as jnp
import numpy as np
from jax.experimental import pallas as pl

_F32 = jnp.float32
_BIAS = float(np.log(np.expm1(1.0)))


def _adj(edge_ref, n_src, n_dst):
    e = edge_ref[...]
    src = e[0, :]
    dst = e[1, :]
    n_e = src.shape[0]
    s_oh = (src[:, None] == jax.lax.broadcasted_iota(jnp.int32, (n_e, n_src), 1)
            ).astype(_F32)
    d_oh = (dst[:, None] == jax.lax.broadcasted_iota(jnp.int32, (n_e, n_dst), 1)
            ).astype(_F32)
    return jax.lax.dot_general(
        d_oh, s_oh, (((0,), (0,)), ((), ())), preferred_element_type=_F32)


def _mm(a, b):
    return jax.lax.dot_general(
        a, b, (((1,), (0,)), ((), ())), preferred_element_type=_F32)


def _body(x_joint, x_torso, Wj, bj, Wt, bt,
          W1_tj_rel, b1_tj, W1_tj_root, W1_jj_rel, b1_jj, W1_jj_root,
          W1_jt_rel, b1_jt, W1_jt_root,
          W2_tj_rel, b2_tj, W2_tj_root, W2_jj_rel, b2_jj, W2_jj_root,
          W2_jt_rel, b2_jt, W2_jt_root,
          Wbig, bbig, ei_tj, ei_jt, ei_jj, out_ref):
    h_j = _mm(x_joint[...], Wj[...]) + bj[...][None, :]
    h_t = _mm(x_torso[...], Wt[...]) + bt[...][None, :]

    A_tj = _adj(ei_tj, 10, 80)
    A_jj = _adj(ei_jj, 80, 80)
    A_jt = _adj(ei_jt, 80, 10)

    j1 = (_mm(_mm(A_tj, h_t), W1_tj_rel[...]) + b1_tj[...][None, :]
          + _mm(_mm(A_jj, h_j), W1_jj_rel[...]) + b1_jj[...][None, :]
          + _mm(h_j, W1_tj_root[...] + W1_jj_root[...]))
    t1 = (_mm(_mm(A_jt, h_j), W1_jt_rel[...]) + b1_jt[...][None, :]
          + _mm(h_t, W1_jt_root[...]))
    j1 = jnp.tanh(j1)
    t1 = jnp.tanh(t1)

    j2 = (_mm(_mm(A_tj, t1), W2_tj_rel[...]) + b2_tj[...][None, :]
          + _mm(_mm(A_jj, j1), W2_jj_rel[...]) + b2_jj[...][None, :]
          + _mm(j1, W2_tj_root[...] + W2_jj_root[...]))
    j2 = jnp.tanh(j2)

    out16 = _mm(j2, Wbig[...]) + bbig[...][None, :]
    col = jax.lax.broadcasted_iota(jnp.int32, (80, 16), 1)
    head2 = 2 * (jax.lax.broadcasted_iota(jnp.int32, (80, 16), 0) % 8)
    loc = jnp.sum(jnp.where(col == head2, out16, 0.0), axis=1)
    pre = jnp.sum(jnp.where(col == head2 + 1, out16, 0.0), axis=1)
    scale = jnp.maximum(jax.nn.softplus(pre + _BIAS), 1e-4)
    out_ref[0, :] = loc
    out_ref[1, :] = scale


def kernel(x_joint, x_torso, Wj, bj, Wt, bt,
           W1_tj_rel, b1_tj, W1_tj_root, W1_jj_rel, b1_jj, W1_jj_root,
           W1_jt_rel, b1_jt, W1_jt_root,
           W2_tj_rel, b2_tj, W2_tj_root, W2_jj_rel, b2_jj, W2_jj_root,
           W2_jt_rel, b2_jt, W2_jt_root,
           Wout, bout, ei_tj, ei_jt, ei_jj):
    Wbig = jnp.transpose(Wout, (1, 0, 2)).reshape(64, 16)
    bbig = bout.reshape(16)
    out = pl.pallas_call(
        _body,
        out_shape=jax.ShapeDtypeStruct((2, 80), _F32),
    )(x_joint, x_torso, Wj, bj, Wt, bt,
      W1_tj_rel, b1_tj, W1_tj_root, W1_jj_rel, b1_jj, W1_jj_root,
      W1_jt_rel, b1_jt, W1_jt_root,
      W2_tj_rel, b2_tj, W2_tj_root, W2_jj_rel, b2_jj, W2_jj_root,
      W2_jt_rel, b2_jt, W2_jt_root,
      Wbig, bbig, ei_tj.astype(jnp.int32), ei_jt.astype(jnp.int32),
      ei_jj.astype(jnp.int32))
    loc = out[0, :].reshape(10, 8)
    scale = out[1, :].reshape(10, 8)
    return (loc, scale)
